# baseline (speedup 1.0000x reference)
import functools

import jax
import jax.numpy as jnp
from jax import lax
from jax.experimental import pallas as pl
from jax.experimental.pallas import tpu as pltpu
from jax.experimental.pallas import tpu_sc as plsc

BATCH = 4096
HIST = 200
DIM = 32
TILE = 128
NBUF = 8


def _make_gather():
    info = plsc.get_sparse_core_info()
    nc, ns = info.num_cores, info.num_subcores
    nw = nc * ns
    assert BATCH // TILE == nw
    hr_n = HIST // 8
    n_groups = HIST // NBUF

    mesh = plsc.VectorSubcoreMesh(core_axis_name="c", subcore_axis_name="s")

    @functools.partial(
        pl.kernel,
        mesh=mesh,
        out_type=jax.ShapeDtypeStruct((HIST, 4, nw, 8, TILE), jnp.float32),
        scratch_types=[
            pltpu.VMEM((hr_n, 8, TILE), jnp.int32),
            [pltpu.VMEM((TILE, DIM), jnp.float32) for _ in range(NBUF)],
            [pltpu.VMEM((DIM, TILE + 1), jnp.float32) for _ in range(NBUF)],
            pltpu.SemaphoreType.DMA,
            [pltpu.SemaphoreType.DMA for _ in range(NBUF)],
            [pltpu.SemaphoreType.DMA for _ in range(NBUF)],
        ],
        compiler_params=pltpu.CompilerParams(
            use_tc_tiling_on_sc=False, needs_layout_passes=False
        ),
    )
    def gather_kernel(idx_hbm, table_hbm, out_hbm, idx_v, gbufs, tbufs, isem, gsems, ssems):
        wid = lax.axis_index("s") * nc + lax.axis_index("c")

        for hr in range(hr_n):
            pltpu.async_copy(idx_hbm.at[hr, wid], idx_v.at[hr], isem)
        for hr in range(hr_n):
            pltpu.make_async_copy(idx_hbm.at[hr, wid], idx_v.at[hr], isem).wait()

        lane = lax.iota(jnp.int32, 16)
        zero = lane * 0

        def start_gather(h, b):
            pltpu.async_copy(
                table_hbm.at[idx_v.at[h // 8, h % 8]], gbufs[b], gsems[b]
            )

        def wait_gather(h, b):
            pltpu.make_async_copy(
                table_hbm.at[idx_v.at[h // 8, h % 8]], gbufs[b], gsems[b]
            ).wait()

        def transpose(b):
            def step(o4, carry):
                for u in range(4):
                    o2 = o4 * 4 + u
                    col = zero + o2
                    for dd in range(2):
                        v = gbufs[b][o2, pl.ds(dd * 16, 16)]
                        plsc.store_scatter(tbufs[b], [lane + dd * 16, col], v)
                return carry

            lax.fori_loop(0, TILE // 4, step, 0)

        def start_store(h, b):
            for dr in range(4):
                pltpu.async_copy(
                    tbufs[b].at[pl.ds(dr * 8, 8), pl.ds(0, TILE)],
                    out_hbm.at[h, dr, wid],
                    ssems[b],
                )

        def wait_store(h, b):
            for dr in range(4):
                pltpu.make_async_copy(
                    tbufs[b].at[pl.ds(dr * 8, 8), pl.ds(0, TILE)],
                    out_hbm.at[h, dr, wid],
                    ssems[b],
                ).wait()

        for b in range(NBUF):
            start_gather(b, b)

        for b in range(NBUF):
            wait_gather(b, b)
            transpose(b)
            start_store(b, b)
            start_gather(b + NBUF, b)

        def body(j, carry):
            for b in range(NBUF):
                h = j * NBUF + b
                wait_gather(h, b)
                wait_store(h - NBUF, b)
                transpose(b)
                start_store(h, b)
                start_gather(h + NBUF, b)
            return carry

        lax.fori_loop(1, n_groups - 1, body, 0)

        for b in range(NBUF):
            h = (n_groups - 1) * NBUF + b
            wait_gather(h, b)
            wait_store(h - NBUF, b)
            transpose(b)
            start_store(h, b)

        for b in range(NBUF):
            h = (n_groups - 1) * NBUF + b
            wait_store(h, b)

    return gather_kernel


_gather = _make_gather()


@jax.jit
def kernel(xs, table):
    idx_native = xs.T.reshape(HIST // 8, 8, BATCH // TILE, TILE).transpose(0, 2, 1, 3)
    out5 = _gather(idx_native, table)
    out = out5.transpose(2, 4, 0, 1, 3).reshape(BATCH, HIST, DIM)
    return out


# speedup vs baseline: 1.0071x; 1.0071x over previous
import functools

import jax
import jax.numpy as jnp
from jax import lax
from jax.experimental import pallas as pl
from jax.experimental.pallas import tpu as pltpu
from jax.experimental.pallas import tpu_sc as plsc

BATCH = 4096
HIST = 200
DIM = 32
TILE = 128
NBUF = 4


def _make_gather():
    info = plsc.get_sparse_core_info()
    nc, ns = info.num_cores, info.num_subcores
    nw = nc * ns
    assert BATCH // TILE == nw
    hr_n = HIST // 8
    n_groups = HIST // NBUF

    mesh = plsc.VectorSubcoreMesh(core_axis_name="c", subcore_axis_name="s")

    @functools.partial(
        pl.kernel,
        mesh=mesh,
        out_type=jax.ShapeDtypeStruct((HIST, 4, nw, 8, TILE), jnp.float32),
        scratch_types=[
            pltpu.VMEM((hr_n, 8, TILE), jnp.int32),
            [pltpu.VMEM((TILE, DIM), jnp.float32) for _ in range(NBUF)],
            [pltpu.VMEM((DIM, TILE + 1), jnp.float32) for _ in range(NBUF)],
            pltpu.SemaphoreType.DMA,
            [pltpu.SemaphoreType.DMA for _ in range(NBUF)],
            [pltpu.SemaphoreType.DMA for _ in range(NBUF)],
        ],
        compiler_params=pltpu.CompilerParams(
            use_tc_tiling_on_sc=False, needs_layout_passes=False
        ),
    )
    def gather_kernel(idx_hbm, table_hbm, out_hbm, idx_v, gbufs, tbufs, isem, gsems, ssems):
        wid = lax.axis_index("s") * nc + lax.axis_index("c")

        for hr in range(hr_n):
            pltpu.async_copy(idx_hbm.at[hr, wid], idx_v.at[hr], isem)
        for hr in range(hr_n):
            pltpu.make_async_copy(idx_hbm.at[hr, wid], idx_v.at[hr], isem).wait()

        lane = lax.iota(jnp.int32, 16)
        zero = lane * 0

        def start_gather(h, b):
            pltpu.async_copy(
                table_hbm.at[idx_v.at[h // 8, h % 8]], gbufs[b], gsems[b]
            )

        def wait_gather(h, b):
            pltpu.make_async_copy(
                table_hbm.at[idx_v.at[h // 8, h % 8]], gbufs[b], gsems[b]
            ).wait()

        def transpose(b):
            def step(o4, carry):
                for u in range(4):
                    o2 = o4 * 4 + u
                    col = zero + o2
                    for dd in range(2):
                        v = gbufs[b][o2, pl.ds(dd * 16, 16)]
                        plsc.store_scatter(tbufs[b], [lane + dd * 16, col], v)
                return carry

            lax.fori_loop(0, TILE // 4, step, 0)

        def start_store(h, b):
            for dr in range(4):
                pltpu.async_copy(
                    tbufs[b].at[pl.ds(dr * 8, 8), pl.ds(0, TILE)],
                    out_hbm.at[h, dr, wid],
                    ssems[b],
                )

        def wait_store(h, b):
            for dr in range(4):
                pltpu.make_async_copy(
                    tbufs[b].at[pl.ds(dr * 8, 8), pl.ds(0, TILE)],
                    out_hbm.at[h, dr, wid],
                    ssems[b],
                ).wait()

        for b in range(NBUF):
            start_gather(b, b)

        for b in range(NBUF):
            wait_gather(b, b)
            transpose(b)
            start_store(b, b)
            start_gather(b + NBUF, b)

        def body(j, carry):
            for b in range(NBUF):
                h = j * NBUF + b
                wait_gather(h, b)
                wait_store(h - NBUF, b)
                transpose(b)
                start_store(h, b)
                start_gather(h + NBUF, b)
            return carry

        lax.fori_loop(1, n_groups - 1, body, 0)

        for b in range(NBUF):
            h = (n_groups - 1) * NBUF + b
            wait_gather(h, b)
            wait_store(h - NBUF, b)
            transpose(b)
            start_store(h, b)

        for b in range(NBUF):
            h = (n_groups - 1) * NBUF + b
            wait_store(h, b)

    return gather_kernel


_gather = _make_gather()


@jax.jit
def kernel(xs, table):
    idx_native = xs.T.reshape(HIST // 8, 8, BATCH // TILE, TILE).transpose(0, 2, 1, 3)
    out5 = _gather(idx_native, table)
    out = out5.transpose(2, 4, 0, 1, 3).reshape(BATCH, HIST, DIM)
    return out
